# baseline (speedup 1.0000x reference)
import jax
import jax.numpy as jnp
from jax.experimental import pallas as pl
from jax.experimental.pallas import tpu as pltpu

N = 1024
VEC = 256
OUT = 64
EDGE = 4
IB = 32
GRID = N // IB

_NT = (((1,), (1,)), ((), ()))


def _body(x_ref, adj_ref, fin_ref, fout_ref, wnit_ref, wnot_ref,
          weit_ref, weot_ref, waggt_ref, bias_ref,
          out_ref, min_ref, mout_ref,
          sint_s, soutt_s, eins_s, eouta_s):
    i = pl.program_id(0)
    rows = pl.ds(i * IB, IB)

    @pl.when(i == 0)
    def _init():
        xv = x_ref[...]
        sint_s[...] = jax.lax.dot_general(
            wnit_ref[...], xv, _NT, preferred_element_type=jnp.float32)
        soutt_s[...] = jax.lax.dot_general(
            wnot_ref[...], xv, _NT, preferred_element_type=jnp.float32)
        eouta_s[...] = jnp.zeros_like(eouta_s)

    weit = weit_ref[...]
    weot = weot_ref[...]

    fin = fin_ref[...]
    for k in range(IB):
        min_ref[k] = jnp.dot(weit, fin[k], preferred_element_type=jnp.float32)
    eins_s[rows, :] = fin.sum(axis=2)

    fout = fout_ref[...]
    for k in range(IB):
        mout_ref[k] = jnp.dot(weot, fout[k], preferred_element_type=jnp.float32)
    eouta_s[...] += fout.sum(axis=0)

    @pl.when(i == GRID - 1)
    def _finish():
        adj = adj_ref[...]
        nin_t = jnp.dot(sint_s[...], adj, preferred_element_type=jnp.float32)
        nout_t = jax.lax.dot_general(
            soutt_s[...], adj, _NT, preferred_element_type=jnp.float32)
        eins_t = jax.lax.dot_general(
            weit, eins_s[...], _NT, preferred_element_type=jnp.float32)
        eout_t = jnp.dot(weot, eouta_s[...], preferred_element_type=jnp.float32)
        waggt = waggt_ref[...]
        h = OUT // 2
        acc = jnp.dot(waggt[:, 0:h], nin_t, preferred_element_type=jnp.float32)
        acc += jnp.dot(waggt[:, h:2 * h], nout_t,
                       preferred_element_type=jnp.float32)
        acc += jnp.dot(waggt[:, 2 * h:2 * h + OUT], eins_t,
                       preferred_element_type=jnp.float32)
        acc += jnp.dot(waggt[:, 2 * h + OUT:], eout_t,
                       preferred_element_type=jnp.float32)
        out_ref[...] = jnp.maximum(acc + bias_ref[...], 0.0)


@jax.jit
def kernel(x, adj_matrix, edge_in_feat_matrix, edge_out_feat_matrix,
           weight_node_in, weight_node_out, weight_edge_in, weight_edge_out,
           weight_aggressive, bias):
    fin_t = jnp.transpose(edge_in_feat_matrix, (0, 2, 1))
    fout_t = jnp.transpose(edge_out_feat_matrix, (0, 2, 1))
    wni_t = weight_node_in.T
    wno_t = weight_node_out.T
    wei_t = weight_edge_in.T
    weo_t = weight_edge_out.T
    wagg_t = weight_aggressive.T
    bias_c = bias.reshape(OUT, 1)

    in_specs = [
        pl.BlockSpec((N, VEC), lambda i: (0, 0)),
        pl.BlockSpec((N, N), lambda i: (0, 0)),
        pl.BlockSpec((IB, EDGE, N), lambda i: (i, 0, 0)),
        pl.BlockSpec((IB, EDGE, N), lambda i: (i, 0, 0)),
        pl.BlockSpec((OUT // 2, VEC), lambda i: (0, 0)),
        pl.BlockSpec((OUT // 2, VEC), lambda i: (0, 0)),
        pl.BlockSpec((OUT, EDGE), lambda i: (0, 0)),
        pl.BlockSpec((OUT, EDGE), lambda i: (0, 0)),
        pl.BlockSpec((OUT, 3 * OUT), lambda i: (0, 0)),
        pl.BlockSpec((OUT, 1), lambda i: (0, 0)),
    ]
    out_specs = [
        pl.BlockSpec((OUT, N), lambda i: (0, 0)),
        pl.BlockSpec((IB, OUT, N), lambda i: (i, 0, 0)),
        pl.BlockSpec((IB, OUT, N), lambda i: (i, 0, 0)),
    ]

    out_t, min_t, mout_t = pl.pallas_call(
        _body,
        grid=(GRID,),
        in_specs=in_specs,
        out_specs=out_specs,
        out_shape=[
            jax.ShapeDtypeStruct((OUT, N), jnp.float32),
            jax.ShapeDtypeStruct((N, OUT, N), jnp.float32),
            jax.ShapeDtypeStruct((N, OUT, N), jnp.float32),
        ],
        scratch_shapes=[
            pltpu.VMEM((OUT // 2, N), jnp.float32),
            pltpu.VMEM((OUT // 2, N), jnp.float32),
            pltpu.VMEM((N, EDGE), jnp.float32),
            pltpu.VMEM((EDGE, N), jnp.float32),
        ],
        compiler_params=pltpu.CompilerParams(
            dimension_semantics=("arbitrary",),
        ),
    )(x, adj_matrix, fin_t, fout_t, wni_t, wno_t, wei_t, weo_t,
      wagg_t, bias_c)

    return (out_t.T,
            jnp.transpose(min_t, (0, 2, 1)),
            jnp.transpose(mout_t, (0, 2, 1)))
